# baseline (speedup 1.0000x reference)
import functools

import jax
import jax.numpy as jnp
from jax.experimental import pallas as pl
from jax.experimental.pallas import tpu as pltpu

BATCH = 128
VOCAB = 100000
BLOCK_W = 12800
NUM_BLOCKS = (VOCAB + BLOCK_W - 1) // BLOCK_W

_KEY0 = 928981903
_KEY1 = 3453687069


def _u32(x):
    return jnp.uint32(x)


def _rotl(x, d):
    return (x << _u32(d)) | (x >> _u32(32 - d))


def _threefry2x32(x0, x1):
    ks0 = _u32(_KEY0)
    ks1 = _u32(_KEY1)
    ks2 = _u32(_KEY0 ^ _KEY1 ^ 0x1BD11BDA)
    rot0 = (13, 15, 26, 6)
    rot1 = (17, 29, 16, 24)
    x0 = x0 + ks0
    x1 = x1 + ks1
    for rots, ka, kb, inc in (
        (rot0, ks1, ks2, 1),
        (rot1, ks2, ks0, 2),
        (rot0, ks0, ks1, 3),
        (rot1, ks1, ks2, 4),
        (rot0, ks2, ks0, 5),
    ):
        for r in rots:
            x0 = x0 + x1
            x1 = _rotl(x1, r)
            x1 = x1 ^ x0
        x0 = x0 + ka
        x1 = x1 + kb + _u32(inc)
    return x0, x1


def _sample_block(logits_ref, out_ref, best_val, best_idx):
    j = pl.program_id(0)
    shape = (BATCH, BLOCK_W)

    row = jax.lax.broadcasted_iota(jnp.uint32, shape, 0)
    col = jax.lax.broadcasted_iota(jnp.uint32, shape, 1) + _u32(BLOCK_W) * j.astype(
        jnp.uint32
    )
    ctr = row * _u32(VOCAB) + col

    o0, o1 = _threefry2x32(jnp.zeros(shape, jnp.uint32), ctr)
    bits = o0 ^ o1

    fbits = (bits >> _u32(9)) | _u32(0x3F800000)
    f = pltpu.bitcast(fbits, jnp.float32) - jnp.float32(1.0)
    tiny = jnp.float32(jnp.finfo(jnp.float32).tiny)
    u = jnp.maximum(f, tiny)
    g = -jnp.log(-jnp.log(u))

    v = logits_ref[...] + g
    v = jnp.where(col < _u32(VOCAB), v, -jnp.inf)

    local_max = jnp.max(v, axis=1, keepdims=True)
    local_arg = jnp.argmax(v, axis=1).astype(jnp.int32).reshape(BATCH, 1)
    local_arg = local_arg + j * BLOCK_W

    @pl.when(j == 0)
    def _init():
        best_val[...] = jnp.full((BATCH, 1), -jnp.inf, jnp.float32)
        best_idx[...] = jnp.zeros((BATCH, 1), jnp.int32)

    take = local_max > best_val[...]
    best_val[...] = jnp.where(take, local_max, best_val[...])
    best_idx[...] = jnp.where(take, local_arg, best_idx[...])

    @pl.when(j == NUM_BLOCKS - 1)
    def _done():
        out_ref[...] = best_idx[...]


@functools.partial(jax.jit, static_argnames=())
def kernel(logits):
    out = pl.pallas_call(
        _sample_block,
        grid=(NUM_BLOCKS,),
        in_specs=[
            pl.BlockSpec((BATCH, BLOCK_W), lambda j: (0, j)),
        ],
        out_specs=pl.BlockSpec((BATCH, 1), lambda j: (0, 0)),
        out_shape=jax.ShapeDtypeStruct((BATCH, 1), jnp.int32),
        scratch_shapes=[
            pltpu.VMEM((BATCH, 1), jnp.float32),
            pltpu.VMEM((BATCH, 1), jnp.int32),
        ],
        compiler_params=pltpu.CompilerParams(
            dimension_semantics=("arbitrary",),
        ),
    )(logits)
    return out.reshape(BATCH)
